# baseline (speedup 1.0000x reference)
import jax
import jax.numpy as jnp
from jax.experimental import pallas as pl

_B, _P, _T = 8, 16384, 256
_KNN = 819
_STEP = 4
_K2 = 205
_DT = 64
_TBLK = 32


def _fps(pos, K):
    b, p, _ = pos.shape
    def body(i, state):
        idx, dists, farthest = state
        idx = idx.at[:, i].set(farthest)
        centroid = pos[jnp.arange(b), farthest][:, None, :]
        d = jnp.sum((pos - centroid) ** 2, axis=-1)
        dists = jnp.minimum(dists, d)
        farthest = jnp.argmax(dists, axis=-1).astype(jnp.int32)
        return (idx, dists, farthest)
    state = (jnp.zeros((b, K), jnp.int32), jnp.full((b, p), 1e10, pos.dtype),
             jnp.zeros((b,), jnp.int32))
    idx, _, _ = jax.lax.fori_loop(0, K, body, state)
    return idx


def _knn_search(q, t, k):
    d = (jnp.sum(q * q, -1)[:, :, None]
         - 2.0 * jnp.einsum('btd,bpd->btp', q, t)
         + jnp.sum(t * t, -1)[:, None, :])
    _, idx = jax.lax.top_k(-d, k)
    return idx


def _disambiguate(df, vecs):
    proj = jnp.einsum('btkd,btd->btk', df, vecs)
    n_pos = jnp.sum((proj > 0.0).astype(df.dtype), axis=2, keepdims=True)
    flip = (n_pos < 0.5 * df.shape[2]).astype(df.dtype)
    return (1.0 - 2.0 * flip) * vecs


def _rotmlp_body(lps_ref, ctr_ref, lrf_ref, w1_ref, b1_ref, w2_ref, b2_ref,
                 w3_ref, b3_ref, o_ref):
    lps = lps_ref[0]
    ctr = ctr_ref[0]
    lrf = lrf_ref[0]
    w1 = w1_ref[...]
    h1 = jnp.zeros((_TBLK, _K2, _DT), jnp.float32) + b1_ref[...][0][None, None, :]
    for dd in range(3):
        pr = jnp.zeros((_TBLK, _K2), jnp.float32)
        orr = jnp.zeros((_TBLK, _K2), jnp.float32)
        for c in range(3):
            lp = lps[:, :, c] - ctr[:, c][:, None]
            lo = lps[:, :, 3 + c]
            ld = lrf[:, c, dd][:, None]
            pr = pr + lp * ld
            orr = orr + lo * ld
        h1 = h1 + pr[:, :, None] * w1[dd][None, None, :]
        h1 = h1 + orr[:, :, None] * w1[3 + dd][None, None, :]
    h1 = jnp.maximum(h1, 0.0)
    h = jnp.maximum(jnp.dot(h1, w2_ref[...], preferred_element_type=jnp.float32)
                    + b2_ref[...][0][None, None, :], 0.0)
    h = (jnp.dot(h, w3_ref[...], preferred_element_type=jnp.float32)
         + b3_ref[...][0][None, None, :])
    o_ref[0] = jnp.max(h, axis=1)


def _rotmlp(lps, ctr_pos, lrfs, W1, b1, W2, b2, W3, b3):
    return pl.pallas_call(
        _rotmlp_body,
        grid=(_B, _T // _TBLK),
        in_specs=[
            pl.BlockSpec((1, _TBLK, _K2, 6), lambda b, t: (b, t, 0, 0)),
            pl.BlockSpec((1, _TBLK, 3), lambda b, t: (b, t, 0)),
            pl.BlockSpec((1, _TBLK, 3, 3), lambda b, t: (b, t, 0, 0)),
            pl.BlockSpec((6, _DT), lambda b, t: (0, 0)),
            pl.BlockSpec((1, _DT), lambda b, t: (0, 0)),
            pl.BlockSpec((_DT, _DT), lambda b, t: (0, 0)),
            pl.BlockSpec((1, _DT), lambda b, t: (0, 0)),
            pl.BlockSpec((_DT, _DT), lambda b, t: (0, 0)),
            pl.BlockSpec((1, _DT), lambda b, t: (0, 0)),
        ],
        out_specs=pl.BlockSpec((1, _TBLK, _DT), lambda b, t: (b, t, 0)),
        out_shape=jax.ShapeDtypeStruct((_B, _T, _DT), jnp.float32),
    )(lps, ctr_pos, lrfs, W1, b1.reshape(1, _DT), W2, b2.reshape(1, _DT),
      W3, b3.reshape(1, _DT))


def kernel(input, W1, b1, W2, b2, W3, b3):
    pos = input[..., 0:3]
    rep_idx = _fps(pos, _T)
    gidx = jnp.repeat(rep_idx[:, :, None], 3, axis=2)
    ctr_pos = jnp.take_along_axis(pos, gidx, axis=1)
    nn_idx = _knn_search(ctr_pos, pos, _KNN)[:, :, ::_STEP]
    lps = jax.vmap(lambda tt, ii: tt[ii])(input, nn_idx)
    lps_pos = lps[..., 0:3] - ctr_pos[:, :, None, :]
    norms = jnp.linalg.norm(lps_pos, axis=3, keepdims=True)
    max_norms = jnp.max(norms, axis=2, keepdims=True)
    w = max_norms - norms
    w = w / jnp.sum(w, axis=2, keepdims=True)
    scaled = 100.0 * lps_pos
    covs = jnp.einsum('bijk,bijl->bikl', w * scaled, scaled)
    _, evecs = jnp.linalg.eigh(covs)
    n = _disambiguate(lps_pos, evecs[:, :, :, 0])
    z = _disambiguate(lps_pos, evecs[:, :, :, 2])
    y = jnp.cross(n, z, axis=-1)
    trf = jnp.stack((n, y, z), axis=3)
    tokens = _rotmlp(lps, ctr_pos, trf, W1, b1, W2, b2, W3, b3)
    return (tokens, ctr_pos, trf)


# speedup vs baseline: 1.0484x; 1.0484x over previous
import jax
import jax.numpy as jnp
from jax.experimental import pallas as pl

_B, _P, _T = 8, 16384, 256
_KNN = 819
_STEP = 4
_K2 = 205
_DT = 64
_TBLK = 32


def _fps(pos, K):
    b, p, _ = pos.shape
    def body(i, state):
        idx, dists, farthest = state
        idx = idx.at[:, i].set(farthest)
        centroid = pos[jnp.arange(b), farthest][:, None, :]
        d = jnp.sum((pos - centroid) ** 2, axis=-1)
        dists = jnp.minimum(dists, d)
        farthest = jnp.argmax(dists, axis=-1).astype(jnp.int32)
        return (idx, dists, farthest)
    state = (jnp.zeros((b, K), jnp.int32), jnp.full((b, p), 1e10, pos.dtype),
             jnp.zeros((b,), jnp.int32))
    idx, _, _ = jax.lax.fori_loop(0, K, body, state)
    return idx


def _knn_search(q, t, k):
    d = (jnp.sum(q * q, -1)[:, :, None]
         - 2.0 * jnp.einsum('btd,bpd->btp', q, t)
         + jnp.sum(t * t, -1)[:, None, :])
    _, idx = jax.lax.top_k(-d, k)
    return idx


def _disambiguate(df, vecs):
    proj = jnp.einsum('btkd,btd->btk', df, vecs)
    n_pos = jnp.sum((proj > 0.0).astype(df.dtype), axis=2, keepdims=True)
    flip = (n_pos < 0.5 * df.shape[2]).astype(df.dtype)
    return (1.0 - 2.0 * flip) * vecs


_K2P = 208


def _rotmlp_body(lps_ref, ctr_ref, lrf_ref, w1_ref, b1_ref, w2_ref, b2_ref,
                 w3_ref, b3_ref, o_ref):
    x = lps_ref[0]
    ctr = ctr_ref[0]
    lrf = lrf_ref[0]
    w1 = w1_ref[...]
    h1 = jnp.zeros((_TBLK, _K2P, _DT), jnp.float32) + b1_ref[...][0][None, None, :]
    for dd in range(3):
        pr = jnp.zeros((_TBLK, _K2P), jnp.float32)
        orr = jnp.zeros((_TBLK, _K2P), jnp.float32)
        for c in range(3):
            lp = x[c] - ctr[:, c][:, None]
            lo = x[3 + c]
            ld = lrf[:, c, dd][:, None]
            pr = pr + lp * ld
            orr = orr + lo * ld
        h1 = h1 + pr[:, :, None] * w1[dd][None, None, :]
        h1 = h1 + orr[:, :, None] * w1[3 + dd][None, None, :]
    h1 = jnp.maximum(h1, 0.0).reshape(_TBLK * _K2P, _DT)
    h = jnp.maximum(jnp.dot(h1, w2_ref[...], preferred_element_type=jnp.float32)
                    + b2_ref[...], 0.0)
    h = (jnp.dot(h, w3_ref[...], preferred_element_type=jnp.float32)
         + b3_ref[...]).reshape(_TBLK, _K2P, _DT)
    mask = jax.lax.broadcasted_iota(jnp.int32, (1, _K2P, 1), 1) < _K2
    o_ref[0] = jnp.max(jnp.where(mask, h, -jnp.inf), axis=1)


def _rotmlp(lps, ctr_pos, lrfs, W1, b1, W2, b2, W3, b3):
    lpsT = jnp.pad(lps.transpose(0, 3, 1, 2),
                   ((0, 0), (0, 0), (0, 0), (0, _K2P - _K2)))
    return pl.pallas_call(
        _rotmlp_body,
        grid=(_B, _T // _TBLK),
        in_specs=[
            pl.BlockSpec((1, 6, _TBLK, _K2P), lambda b, t: (b, 0, t, 0)),
            pl.BlockSpec((1, _TBLK, 3), lambda b, t: (b, t, 0)),
            pl.BlockSpec((1, _TBLK, 3, 3), lambda b, t: (b, t, 0, 0)),
            pl.BlockSpec((6, _DT), lambda b, t: (0, 0)),
            pl.BlockSpec((1, _DT), lambda b, t: (0, 0)),
            pl.BlockSpec((_DT, _DT), lambda b, t: (0, 0)),
            pl.BlockSpec((1, _DT), lambda b, t: (0, 0)),
            pl.BlockSpec((_DT, _DT), lambda b, t: (0, 0)),
            pl.BlockSpec((1, _DT), lambda b, t: (0, 0)),
        ],
        out_specs=pl.BlockSpec((1, _TBLK, _DT), lambda b, t: (b, t, 0)),
        out_shape=jax.ShapeDtypeStruct((_B, _T, _DT), jnp.float32),
    )(lpsT, ctr_pos, lrfs, W1, b1.reshape(1, _DT), W2, b2.reshape(1, _DT),
      W3, b3.reshape(1, _DT))


def kernel(input, W1, b1, W2, b2, W3, b3):
    pos = input[..., 0:3]
    rep_idx = _fps(pos, _T)
    gidx = jnp.repeat(rep_idx[:, :, None], 3, axis=2)
    ctr_pos = jnp.take_along_axis(pos, gidx, axis=1)
    nn_idx = _knn_search(ctr_pos, pos, _KNN)[:, :, ::_STEP]
    lps = jax.vmap(lambda tt, ii: tt[ii])(input, nn_idx)
    lps_pos = lps[..., 0:3] - ctr_pos[:, :, None, :]
    norms = jnp.linalg.norm(lps_pos, axis=3, keepdims=True)
    max_norms = jnp.max(norms, axis=2, keepdims=True)
    w = max_norms - norms
    w = w / jnp.sum(w, axis=2, keepdims=True)
    scaled = 100.0 * lps_pos
    covs = jnp.einsum('bijk,bijl->bikl', w * scaled, scaled)
    _, evecs = jnp.linalg.eigh(covs)
    n = _disambiguate(lps_pos, evecs[:, :, :, 0])
    z = _disambiguate(lps_pos, evecs[:, :, :, 2])
    y = jnp.cross(n, z, axis=-1)
    trf = jnp.stack((n, y, z), axis=3)
    tokens = _rotmlp(lps, ctr_pos, trf, W1, b1, W2, b2, W3, b3)
    return (tokens, ctr_pos, trf)


# speedup vs baseline: 1.3087x; 1.2483x over previous
import jax
import jax.numpy as jnp
from jax.experimental import pallas as pl

_B, _P, _T = 8, 16384, 256
_KNN = 819
_STEP = 4
_K2 = 205
_DT = 64
_TBLK = 32


_PR = 128
_PC = 128


def _fps_body(pos_ref, out_ref):
    x = pos_ref[:, 0]
    y = pos_ref[:, 1]
    z = pos_ref[:, 2]
    iota = (jax.lax.broadcasted_iota(jnp.int32, (_B, _PR, _PC), 1) * _PC
            + jax.lax.broadcasted_iota(jnp.int32, (_B, _PR, _PC), 2))

    def _red2(v, op):
        return op(op(v, axis=2, keepdims=True), axis=1, keepdims=True)

    def body(i, state):
        dists, farr = state
        out_ref[pl.ds(i, 1), :] = farr.reshape(1, _B)
        sel = iota == farr[:, None, None]
        cx = _red2(jnp.where(sel, x, 0.0), jnp.sum)
        cy = _red2(jnp.where(sel, y, 0.0), jnp.sum)
        cz = _red2(jnp.where(sel, z, 0.0), jnp.sum)
        dx = x - cx
        dy = y - cy
        dz = z - cz
        d = (dx * dx + dy * dy) + dz * dz
        dists = jnp.minimum(dists, d)
        m = _red2(dists, jnp.max)
        idxm = _red2(jnp.where(dists == m, iota, _P), jnp.min)
        return dists, idxm[:, 0, 0]

    state = (jnp.full((_B, _PR, _PC), 1e10, jnp.float32),
             jnp.zeros((_B,), jnp.int32))
    jax.lax.fori_loop(0, _T, body, state)


def _fps(pos, K):
    posR = pos.transpose(0, 2, 1).reshape(_B, 3, _PR, _PC)
    out = pl.pallas_call(
        _fps_body,
        in_specs=[pl.BlockSpec((_B, 3, _PR, _PC), lambda: (0, 0, 0, 0))],
        out_specs=pl.BlockSpec((_T, _B), lambda: (0, 0)),
        out_shape=jax.ShapeDtypeStruct((_T, _B), jnp.int32),
    )(posR)
    return out.T


def _knn_search(q, t, k):
    d = (jnp.sum(q * q, -1)[:, :, None]
         - 2.0 * jnp.einsum('btd,bpd->btp', q, t)
         + jnp.sum(t * t, -1)[:, None, :])
    _, idx = jax.lax.top_k(-d, k)
    return idx


def _disambiguate(df, vecs):
    proj = jnp.einsum('btkd,btd->btk', df, vecs)
    n_pos = jnp.sum((proj > 0.0).astype(df.dtype), axis=2, keepdims=True)
    flip = (n_pos < 0.5 * df.shape[2]).astype(df.dtype)
    return (1.0 - 2.0 * flip) * vecs


_K2P = 208


def _rotmlp_body(lps_ref, ctr_ref, lrf_ref, w1_ref, b1_ref, w2_ref, b2_ref,
                 w3_ref, b3_ref, o_ref):
    x = lps_ref[0]
    ctr = ctr_ref[0]
    lrf = lrf_ref[0]
    w1 = w1_ref[...]
    h1 = jnp.zeros((_TBLK, _K2P, _DT), jnp.float32) + b1_ref[...][0][None, None, :]
    for dd in range(3):
        pr = jnp.zeros((_TBLK, _K2P), jnp.float32)
        orr = jnp.zeros((_TBLK, _K2P), jnp.float32)
        for c in range(3):
            lp = x[c] - ctr[:, c][:, None]
            lo = x[3 + c]
            ld = lrf[:, c, dd][:, None]
            pr = pr + lp * ld
            orr = orr + lo * ld
        h1 = h1 + pr[:, :, None] * w1[dd][None, None, :]
        h1 = h1 + orr[:, :, None] * w1[3 + dd][None, None, :]
    h1 = jnp.maximum(h1, 0.0).reshape(_TBLK * _K2P, _DT)
    h = jnp.maximum(jnp.dot(h1, w2_ref[...], preferred_element_type=jnp.float32)
                    + b2_ref[...], 0.0)
    h = (jnp.dot(h, w3_ref[...], preferred_element_type=jnp.float32)
         + b3_ref[...]).reshape(_TBLK, _K2P, _DT)
    mask = jax.lax.broadcasted_iota(jnp.int32, (1, _K2P, 1), 1) < _K2
    o_ref[0] = jnp.max(jnp.where(mask, h, -jnp.inf), axis=1)


def _rotmlp(lps, ctr_pos, lrfs, W1, b1, W2, b2, W3, b3):
    lpsT = jnp.pad(lps.transpose(0, 3, 1, 2),
                   ((0, 0), (0, 0), (0, 0), (0, _K2P - _K2)))
    return pl.pallas_call(
        _rotmlp_body,
        grid=(_B, _T // _TBLK),
        in_specs=[
            pl.BlockSpec((1, 6, _TBLK, _K2P), lambda b, t: (b, 0, t, 0)),
            pl.BlockSpec((1, _TBLK, 3), lambda b, t: (b, t, 0)),
            pl.BlockSpec((1, _TBLK, 3, 3), lambda b, t: (b, t, 0, 0)),
            pl.BlockSpec((6, _DT), lambda b, t: (0, 0)),
            pl.BlockSpec((1, _DT), lambda b, t: (0, 0)),
            pl.BlockSpec((_DT, _DT), lambda b, t: (0, 0)),
            pl.BlockSpec((1, _DT), lambda b, t: (0, 0)),
            pl.BlockSpec((_DT, _DT), lambda b, t: (0, 0)),
            pl.BlockSpec((1, _DT), lambda b, t: (0, 0)),
        ],
        out_specs=pl.BlockSpec((1, _TBLK, _DT), lambda b, t: (b, t, 0)),
        out_shape=jax.ShapeDtypeStruct((_B, _T, _DT), jnp.float32),
    )(lpsT, ctr_pos, lrfs, W1, b1.reshape(1, _DT), W2, b2.reshape(1, _DT),
      W3, b3.reshape(1, _DT))


def kernel(input, W1, b1, W2, b2, W3, b3):
    pos = input[..., 0:3]
    rep_idx = _fps(pos, _T)
    gidx = jnp.repeat(rep_idx[:, :, None], 3, axis=2)
    ctr_pos = jnp.take_along_axis(pos, gidx, axis=1)
    nn_idx = _knn_search(ctr_pos, pos, _KNN)[:, :, ::_STEP]
    lps = jax.vmap(lambda tt, ii: tt[ii])(input, nn_idx)
    lps_pos = lps[..., 0:3] - ctr_pos[:, :, None, :]
    norms = jnp.linalg.norm(lps_pos, axis=3, keepdims=True)
    max_norms = jnp.max(norms, axis=2, keepdims=True)
    w = max_norms - norms
    w = w / jnp.sum(w, axis=2, keepdims=True)
    scaled = 100.0 * lps_pos
    covs = jnp.einsum('bijk,bijl->bikl', w * scaled, scaled)
    _, evecs = jnp.linalg.eigh(covs)
    n = _disambiguate(lps_pos, evecs[:, :, :, 0])
    z = _disambiguate(lps_pos, evecs[:, :, :, 2])
    y = jnp.cross(n, z, axis=-1)
    trf = jnp.stack((n, y, z), axis=3)
    tokens = _rotmlp(lps, ctr_pos, trf, W1, b1, W2, b2, W3, b3)
    return (tokens, ctr_pos, trf)


# speedup vs baseline: 1.7980x; 1.3739x over previous
import functools

import jax
import jax.numpy as jnp
from jax import lax
from jax.experimental import pallas as pl
from jax.experimental.pallas import tpu as pltpu
from jax.experimental.pallas import tpu_sc as plsc

_B, _P, _T = 8, 16384, 256
_KNN = 819
_STEP = 4
_K2 = 205
_DT = 64
_TBLK = 32


_PR = 128
_PC = 128


def _fps_body(pos_ref, out_ref):
    x = pos_ref[:, 0]
    y = pos_ref[:, 1]
    z = pos_ref[:, 2]
    iota = (jax.lax.broadcasted_iota(jnp.int32, (_B, _PR, _PC), 1) * _PC
            + jax.lax.broadcasted_iota(jnp.int32, (_B, _PR, _PC), 2))

    def _red2(v, op):
        return op(op(v, axis=2, keepdims=True), axis=1, keepdims=True)

    def body(i, state):
        dists, farr = state
        out_ref[pl.ds(i, 1), :] = farr.reshape(1, _B)
        sel = iota == farr[:, None, None]
        cx = _red2(jnp.where(sel, x, 0.0), jnp.sum)
        cy = _red2(jnp.where(sel, y, 0.0), jnp.sum)
        cz = _red2(jnp.where(sel, z, 0.0), jnp.sum)
        dx = x - cx
        dy = y - cy
        dz = z - cz
        d = (dx * dx + dy * dy) + dz * dz
        dists = jnp.minimum(dists, d)
        m = _red2(dists, jnp.max)
        idxm = _red2(jnp.where(dists == m, iota, _P), jnp.min)
        return dists, idxm[:, 0, 0]

    state = (jnp.full((_B, _PR, _PC), 1e10, jnp.float32),
             jnp.zeros((_B,), jnp.int32))
    jax.lax.fori_loop(0, _T, body, state)


def _fps(pos, K):
    posR = pos.transpose(0, 2, 1).reshape(_B, 3, _PR, _PC)
    out = pl.pallas_call(
        _fps_body,
        in_specs=[pl.BlockSpec((_B, 3, _PR, _PC), lambda: (0, 0, 0, 0))],
        out_specs=pl.BlockSpec((_T, _B), lambda: (0, 0)),
        out_shape=jax.ShapeDtypeStruct((_T, _B), jnp.int32),
    )(posR)
    return out.T


_NW = 32
_GJ = 103
_GC = 128


def _sc_gather(tbl, idx3):
    mesh = plsc.VectorSubcoreMesh(core_axis_name="c", subcore_axis_name="s")

    @functools.partial(
        pl.kernel,
        out_type=jax.ShapeDtypeStruct((_NW * _GJ * _GC, 16), jnp.float32),
        mesh=mesh,
        scratch_types=[
            pltpu.VMEM((_GJ, _GC), jnp.int32),
            pltpu.VMEM((_GC, 16), jnp.float32),
            pltpu.SemaphoreType.DMA,
        ],
        compiler_params=pltpu.CompilerParams(use_tc_tiling_on_sc=False),
    )
    def gk(tbl_hbm, idx_hbm, out_hbm, idx_v, rows_v, sem):
        wid = lax.axis_index("s") * 2 + lax.axis_index("c")
        pltpu.sync_copy(idx_hbm.at[wid], idx_v)
        base = wid * (_GJ * _GC)

        def body(j, _):
            pltpu.async_copy(tbl_hbm.at[idx_v.at[j]], rows_v, sem).wait()
            pltpu.sync_copy(rows_v, out_hbm.at[pl.ds(base + j * _GC, _GC)])
            return 0

        lax.fori_loop(0, _GJ, body, 0)

    return gk(tbl, idx3)


def _gather_lps(input, nn_idx):
    tbl = jnp.pad(input.reshape(_B * _P, 6), ((0, 0), (0, 10)))
    flat = (nn_idx
            + (jnp.arange(_B, dtype=jnp.int32) * _P)[:, None, None]).reshape(-1)
    flat = jnp.pad(flat, (0, _NW * _GJ * _GC - flat.size))
    g = _sc_gather(tbl, flat.reshape(_NW, _GJ, _GC))
    return g[: _B * _T * _K2, 0:6].reshape(_B, _T, _K2, 6)


def _knn_search(q, t, k):
    d = (jnp.sum(q * q, -1)[:, :, None]
         - 2.0 * jnp.einsum('btd,bpd->btp', q, t)
         + jnp.sum(t * t, -1)[:, None, :])
    _, idx = jax.lax.top_k(-d, k)
    return idx


def _disambiguate(df, vecs):
    proj = jnp.einsum('btkd,btd->btk', df, vecs)
    n_pos = jnp.sum((proj > 0.0).astype(df.dtype), axis=2, keepdims=True)
    flip = (n_pos < 0.5 * df.shape[2]).astype(df.dtype)
    return (1.0 - 2.0 * flip) * vecs


_K2P = 208


def _rotmlp_body(lps_ref, ctr_ref, lrf_ref, w1_ref, b1_ref, w2_ref, b2_ref,
                 w3_ref, b3_ref, o_ref):
    x = lps_ref[0]
    ctr = ctr_ref[0]
    lrf = lrf_ref[0]
    w1 = w1_ref[...]
    h1 = jnp.zeros((_TBLK, _K2P, _DT), jnp.float32) + b1_ref[...][0][None, None, :]
    for dd in range(3):
        pr = jnp.zeros((_TBLK, _K2P), jnp.float32)
        orr = jnp.zeros((_TBLK, _K2P), jnp.float32)
        for c in range(3):
            lp = x[c] - ctr[:, c][:, None]
            lo = x[3 + c]
            ld = lrf[:, c, dd][:, None]
            pr = pr + lp * ld
            orr = orr + lo * ld
        h1 = h1 + pr[:, :, None] * w1[dd][None, None, :]
        h1 = h1 + orr[:, :, None] * w1[3 + dd][None, None, :]
    h1 = jnp.maximum(h1, 0.0).reshape(_TBLK * _K2P, _DT)
    h = jnp.maximum(jnp.dot(h1, w2_ref[...], preferred_element_type=jnp.float32)
                    + b2_ref[...], 0.0)
    h = (jnp.dot(h, w3_ref[...], preferred_element_type=jnp.float32)
         + b3_ref[...]).reshape(_TBLK, _K2P, _DT)
    mask = jax.lax.broadcasted_iota(jnp.int32, (1, _K2P, 1), 1) < _K2
    o_ref[0] = jnp.max(jnp.where(mask, h, -jnp.inf), axis=1)


def _rotmlp(lps, ctr_pos, lrfs, W1, b1, W2, b2, W3, b3):
    lpsT = jnp.pad(lps.transpose(0, 3, 1, 2),
                   ((0, 0), (0, 0), (0, 0), (0, _K2P - _K2)))
    return pl.pallas_call(
        _rotmlp_body,
        grid=(_B, _T // _TBLK),
        in_specs=[
            pl.BlockSpec((1, 6, _TBLK, _K2P), lambda b, t: (b, 0, t, 0)),
            pl.BlockSpec((1, _TBLK, 3), lambda b, t: (b, t, 0)),
            pl.BlockSpec((1, _TBLK, 3, 3), lambda b, t: (b, t, 0, 0)),
            pl.BlockSpec((6, _DT), lambda b, t: (0, 0)),
            pl.BlockSpec((1, _DT), lambda b, t: (0, 0)),
            pl.BlockSpec((_DT, _DT), lambda b, t: (0, 0)),
            pl.BlockSpec((1, _DT), lambda b, t: (0, 0)),
            pl.BlockSpec((_DT, _DT), lambda b, t: (0, 0)),
            pl.BlockSpec((1, _DT), lambda b, t: (0, 0)),
        ],
        out_specs=pl.BlockSpec((1, _TBLK, _DT), lambda b, t: (b, t, 0)),
        out_shape=jax.ShapeDtypeStruct((_B, _T, _DT), jnp.float32),
    )(lpsT, ctr_pos, lrfs, W1, b1.reshape(1, _DT), W2, b2.reshape(1, _DT),
      W3, b3.reshape(1, _DT))


def kernel(input, W1, b1, W2, b2, W3, b3):
    pos = input[..., 0:3]
    rep_idx = _fps(pos, _T)
    gidx = jnp.repeat(rep_idx[:, :, None], 3, axis=2)
    ctr_pos = jnp.take_along_axis(pos, gidx, axis=1)
    nn_idx = _knn_search(ctr_pos, pos, _KNN)[:, :, ::_STEP]
    lps = _gather_lps(input, nn_idx)
    lps_pos = lps[..., 0:3] - ctr_pos[:, :, None, :]
    norms = jnp.linalg.norm(lps_pos, axis=3, keepdims=True)
    max_norms = jnp.max(norms, axis=2, keepdims=True)
    w = max_norms - norms
    w = w / jnp.sum(w, axis=2, keepdims=True)
    scaled = 100.0 * lps_pos
    covs = jnp.einsum('bijk,bijl->bikl', w * scaled, scaled)
    _, evecs = jnp.linalg.eigh(covs)
    n = _disambiguate(lps_pos, evecs[:, :, :, 0])
    z = _disambiguate(lps_pos, evecs[:, :, :, 2])
    y = jnp.cross(n, z, axis=-1)
    trf = jnp.stack((n, y, z), axis=3)
    tokens = _rotmlp(lps, ctr_pos, trf, W1, b1, W2, b2, W3, b3)
    return (tokens, ctr_pos, trf)


# speedup vs baseline: 2.1412x; 1.1908x over previous
import functools

import jax
import jax.numpy as jnp
from jax import lax
from jax.experimental import pallas as pl
from jax.experimental.pallas import tpu as pltpu
from jax.experimental.pallas import tpu_sc as plsc

_B, _P, _T = 8, 16384, 256
_KNN = 819
_STEP = 4
_K2 = 205
_DT = 64
_TBLK = 32


_PR = 128
_PC = 128


def _fps_body(pos_ref, out_ref):
    x = pos_ref[:, 0]
    y = pos_ref[:, 1]
    z = pos_ref[:, 2]
    iota = (jax.lax.broadcasted_iota(jnp.int32, (_B, _PR, _PC), 1) * _PC
            + jax.lax.broadcasted_iota(jnp.int32, (_B, _PR, _PC), 2))

    def _red2(v, op):
        return op(op(v, axis=2, keepdims=True), axis=1, keepdims=True)

    def body(i, state):
        dists, farr = state
        out_ref[pl.ds(i, 1), :] = farr.reshape(1, _B)
        sel = iota == farr[:, None, None]
        cx = _red2(jnp.where(sel, x, 0.0), jnp.sum)
        cy = _red2(jnp.where(sel, y, 0.0), jnp.sum)
        cz = _red2(jnp.where(sel, z, 0.0), jnp.sum)
        dx = x - cx
        dy = y - cy
        dz = z - cz
        d = (dx * dx + dy * dy) + dz * dz
        dists = jnp.minimum(dists, d)
        m = _red2(dists, jnp.max)
        idxm = _red2(jnp.where(dists == m, iota, _P), jnp.min)
        return dists, idxm[:, 0, 0]

    state = (jnp.full((_B, _PR, _PC), 1e10, jnp.float32),
             jnp.zeros((_B,), jnp.int32))
    jax.lax.fori_loop(0, _T, body, state)


def _fps(pos, K):
    posR = pos.transpose(0, 2, 1).reshape(_B, 3, _PR, _PC)
    out = pl.pallas_call(
        _fps_body,
        in_specs=[pl.BlockSpec((_B, 3, _PR, _PC), lambda: (0, 0, 0, 0))],
        out_specs=pl.BlockSpec((_T, _B), lambda: (0, 0)),
        out_shape=jax.ShapeDtypeStruct((_T, _B), jnp.int32),
    )(posR)
    return out.T


_NW = 32
_GJ = 103
_GC = 128


def _sc_gather(tbl, idx3):
    mesh = plsc.VectorSubcoreMesh(core_axis_name="c", subcore_axis_name="s")

    @functools.partial(
        pl.kernel,
        out_type=jax.ShapeDtypeStruct((_NW * _GJ * _GC, 16), jnp.float32),
        mesh=mesh,
        scratch_types=[
            pltpu.VMEM((_GJ, _GC), jnp.int32),
            pltpu.VMEM((_GC, 16), jnp.float32),
            pltpu.SemaphoreType.DMA,
        ],
        compiler_params=pltpu.CompilerParams(use_tc_tiling_on_sc=False),
    )
    def gk(tbl_hbm, idx_hbm, out_hbm, idx_v, rows_v, sem):
        wid = lax.axis_index("s") * 2 + lax.axis_index("c")
        pltpu.sync_copy(idx_hbm.at[wid], idx_v)
        base = wid * (_GJ * _GC)

        def body(j, _):
            pltpu.async_copy(tbl_hbm.at[idx_v.at[j]], rows_v, sem).wait()
            pltpu.sync_copy(rows_v, out_hbm.at[pl.ds(base + j * _GC, _GC)])
            return 0

        lax.fori_loop(0, _GJ, body, 0)

    return gk(tbl, idx3)


def _gather_lps(input, nn_idx):
    tbl = jnp.pad(input.reshape(_B * _P, 6), ((0, 0), (0, 10)))
    flat = (nn_idx
            + (jnp.arange(_B, dtype=jnp.int32) * _P)[:, None, None]).reshape(-1)
    flat = jnp.pad(flat, (0, _NW * _GJ * _GC - flat.size))
    g = _sc_gather(tbl, flat.reshape(_NW, _GJ, _GC))
    return g[: _B * _T * _K2, 0:6].reshape(_B, _T, _K2, 6)


def _knn_subsampled(q, t):
    d = (jnp.sum(q * q, -1)[:, :, None]
         - 2.0 * jnp.einsum('btd,bpd->btp', q, t)
         + jnp.sum(t * t, -1)[:, None, :])
    negv, cidx = jax.lax.approx_max_k(-d, 1024, recall_target=1.0)
    _, gs = jax.lax.sort((-negv, cidx), num_keys=2)
    return gs[:, :, 0:_KNN:_STEP]


def _disambiguate(df, vecs):
    proj = jnp.einsum('btkd,btd->btk', df, vecs)
    n_pos = jnp.sum((proj > 0.0).astype(df.dtype), axis=2, keepdims=True)
    flip = (n_pos < 0.5 * df.shape[2]).astype(df.dtype)
    return (1.0 - 2.0 * flip) * vecs


_K2P = 208


def _rotmlp_body(lps_ref, ctr_ref, lrf_ref, w1_ref, b1_ref, w2_ref, b2_ref,
                 w3_ref, b3_ref, o_ref):
    x = lps_ref[0]
    ctr = ctr_ref[0]
    lrf = lrf_ref[0]
    w1 = w1_ref[...]
    h1 = jnp.zeros((_TBLK, _K2P, _DT), jnp.float32) + b1_ref[...][0][None, None, :]
    for dd in range(3):
        pr = jnp.zeros((_TBLK, _K2P), jnp.float32)
        orr = jnp.zeros((_TBLK, _K2P), jnp.float32)
        for c in range(3):
            lp = x[c] - ctr[:, c][:, None]
            lo = x[3 + c]
            ld = lrf[:, c, dd][:, None]
            pr = pr + lp * ld
            orr = orr + lo * ld
        h1 = h1 + pr[:, :, None] * w1[dd][None, None, :]
        h1 = h1 + orr[:, :, None] * w1[3 + dd][None, None, :]
    h1 = jnp.maximum(h1, 0.0).reshape(_TBLK * _K2P, _DT)
    h = jnp.maximum(jnp.dot(h1, w2_ref[...], preferred_element_type=jnp.float32)
                    + b2_ref[...], 0.0)
    h = (jnp.dot(h, w3_ref[...], preferred_element_type=jnp.float32)
         + b3_ref[...]).reshape(_TBLK, _K2P, _DT)
    mask = jax.lax.broadcasted_iota(jnp.int32, (1, _K2P, 1), 1) < _K2
    o_ref[0] = jnp.max(jnp.where(mask, h, -jnp.inf), axis=1)


def _rotmlp(lps, ctr_pos, lrfs, W1, b1, W2, b2, W3, b3):
    lpsT = jnp.pad(lps.transpose(0, 3, 1, 2),
                   ((0, 0), (0, 0), (0, 0), (0, _K2P - _K2)))
    return pl.pallas_call(
        _rotmlp_body,
        grid=(_B, _T // _TBLK),
        in_specs=[
            pl.BlockSpec((1, 6, _TBLK, _K2P), lambda b, t: (b, 0, t, 0)),
            pl.BlockSpec((1, _TBLK, 3), lambda b, t: (b, t, 0)),
            pl.BlockSpec((1, _TBLK, 3, 3), lambda b, t: (b, t, 0, 0)),
            pl.BlockSpec((6, _DT), lambda b, t: (0, 0)),
            pl.BlockSpec((1, _DT), lambda b, t: (0, 0)),
            pl.BlockSpec((_DT, _DT), lambda b, t: (0, 0)),
            pl.BlockSpec((1, _DT), lambda b, t: (0, 0)),
            pl.BlockSpec((_DT, _DT), lambda b, t: (0, 0)),
            pl.BlockSpec((1, _DT), lambda b, t: (0, 0)),
        ],
        out_specs=pl.BlockSpec((1, _TBLK, _DT), lambda b, t: (b, t, 0)),
        out_shape=jax.ShapeDtypeStruct((_B, _T, _DT), jnp.float32),
    )(lpsT, ctr_pos, lrfs, W1, b1.reshape(1, _DT), W2, b2.reshape(1, _DT),
      W3, b3.reshape(1, _DT))


def kernel(input, W1, b1, W2, b2, W3, b3):
    pos = input[..., 0:3]
    rep_idx = _fps(pos, _T)
    gidx = jnp.repeat(rep_idx[:, :, None], 3, axis=2)
    ctr_pos = jnp.take_along_axis(pos, gidx, axis=1)
    nn_idx = _knn_subsampled(ctr_pos, pos)
    lps = _gather_lps(input, nn_idx)
    lps_pos = lps[..., 0:3] - ctr_pos[:, :, None, :]
    norms = jnp.linalg.norm(lps_pos, axis=3, keepdims=True)
    max_norms = jnp.max(norms, axis=2, keepdims=True)
    w = max_norms - norms
    w = w / jnp.sum(w, axis=2, keepdims=True)
    scaled = 100.0 * lps_pos
    covs = jnp.einsum('bijk,bijl->bikl', w * scaled, scaled)
    _, evecs = jnp.linalg.eigh(covs)
    n = _disambiguate(lps_pos, evecs[:, :, :, 0])
    z = _disambiguate(lps_pos, evecs[:, :, :, 2])
    y = jnp.cross(n, z, axis=-1)
    trf = jnp.stack((n, y, z), axis=3)
    tokens = _rotmlp(lps, ctr_pos, trf, W1, b1, W2, b2, W3, b3)
    return (tokens, ctr_pos, trf)
